# baseline (speedup 1.0000x reference)
import functools

import jax
import jax.numpy as jnp
from jax import lax
from jax.experimental import pallas as pl
from jax.experimental.pallas import tpu as pltpu
from jax.experimental.pallas import tpu_sc as plsc

N_ENT = 10000
NUM_REL = 200
BATCH = 10
N_FACT = 320000
D = 128

NC, NS, L = 2, 16, 16
NW = NC * NS
CK = 48
NCH = 210
NPAD = NW * NCH * CK
NROW = N_ENT + BATCH * NUM_REL
STRIPE = 752


def _sc_gnn(x, rel_feat, packed, w2b):
    mesh = plsc.VectorSubcoreMesh(core_axis_name="c", subcore_axis_name="s")

    def buffer_set():
        return [
            pltpu.VMEM((CK, D), jnp.float32),
            pltpu.VMEM((CK, D), jnp.float32),
            pltpu.VMEM((8 * CK,), jnp.int32),
            pltpu.VMEM((CK * L,), jnp.float32),
            pltpu.VMEM((CK,), jnp.int32),
            pltpu.VMEM((CK,), jnp.int32),
            pltpu.VMEM((CK,), jnp.int32),
            pltpu.VMEM((CK,), jnp.int32),
            pltpu.SemaphoreType.DMA,
            pltpu.SemaphoreType.DMA,
        ]

    @functools.partial(
        pl.kernel,
        out_type=jax.ShapeDtypeStruct((NC, NROW, D), jnp.float32),
        mesh=mesh,
        scratch_types=[pltpu.VMEM_SHARED((NROW, D), jnp.float32)]
        + buffer_set() + buffer_set(),
    )
    def body(x_h, rf_h, pk_h, w2_h, out_h, acc, *bufs):
        sets = (bufs[:10], bufs[10:])
        cid = lax.axis_index("c")
        sid = lax.axis_index("s")
        wid = cid * NS + sid

        def pk_issue(c, bset):
            pk, sem_i = bset[2], bset[8]
            ch = wid * NCH + c
            pltpu.async_copy(pk_h.at[pl.ds(ch * 8 * CK, 8 * CK)], pk, sem_i)

        def wexp_issue(c, bset):
            wexp, sem_i = bset[3], bset[8]
            ch = wid * NCH + c
            pltpu.async_copy(w2_h.at[pl.ds(ch * CK * L, CK * L)], wexp, sem_i)

        def idx_wait(c, bset):
            pk, wexp, sem_i = bset[2], bset[3], bset[8]
            ch = wid * NCH + c
            pltpu.make_async_copy(pk_h.at[pl.ds(ch * 8 * CK, 8 * CK)], pk, sem_i).wait()
            pltpu.make_async_copy(w2_h.at[pl.ds(ch * CK * L, CK * L)], wexp, sem_i).wait()

        def vec_prep(bset):
            pk, riv, rsv, tv, hv = bset[2], bset[4], bset[5], bset[6], bset[7]
            for j in range(CK // L):
                sl = pl.ds(L * j, L)
                r16 = pk[pl.ds(CK + L * j, L)] + pk[pl.ds(2 * CK + L * j, L)] * NUM_REL
                riv[sl] = r16
                rsv[sl] = r16 + N_ENT
                tv[sl] = pk[pl.ds(3 * CK + L * j, L)]
                hv[sl] = pk[pl.ds(L * j, L)]

        def gstart(bset):
            xrows, rrows, riv, hv, sem_g = bset[0], bset[1], bset[4], bset[7], bset[9]
            pltpu.async_copy(x_h.at[hv], xrows, sem_g)
            pltpu.async_copy(rf_h.at[riv], rrows, sem_g)

        def gwait(bset):
            xrows, rrows, riv, hv, sem_g = bset[0], bset[1], bset[4], bset[7], bset[9]
            pltpu.make_async_copy(x_h.at[hv], xrows, sem_g).wait()
            pltpu.make_async_copy(rf_h.at[riv], rrows, sem_g).wait()

        def compute(bset):
            xrows, rrows, wexp = bset[0], bset[1], bset[3]

            def fact(f, c2):
                s = wexp[pl.ds(f * L, L)]
                s2 = s * s
                for j in range(D // L):
                    sl = pl.ds(L * j, L)
                    xrows[f, sl] = (xrows[f, sl] + rrows[f, sl]) * s2
                return c2

            lax.fori_loop(0, CK, fact, 0, unroll=4)

        def scatter(bset):
            xrows, rsv, tv = bset[0], bset[5], bset[6]
            pltpu.sync_copy(xrows, acc.at[tv], add=True)
            pltpu.sync_copy(xrows, acc.at[rsv], add=True)

        xrows0 = sets[0][0]
        zvec = jnp.zeros((L,), jnp.float32)

        def zrow(r, carry):
            for j in range(D // L):
                xrows0[r, pl.ds(L * j, L)] = zvec
            return carry

        lax.fori_loop(0, CK, zrow, 0)
        sbase = jnp.minimum(sid * STRIPE, NROW - STRIPE)
        for kk in range(STRIPE // CK):
            pltpu.sync_copy(xrows0, acc.at[pl.ds(sbase + kk * CK, CK)])
        rem = STRIPE % CK
        if rem:
            pltpu.sync_copy(
                xrows0.at[pl.ds(0, rem)],
                acc.at[pl.ds(sbase + (STRIPE // CK) * CK, rem)],
            )
        plsc.subcore_barrier()

        for half, bset in enumerate(sets):
            pk_issue(half, bset)
            wexp_issue(half, bset)
            idx_wait(half, bset)
            vec_prep(bset)
            gstart(bset)

        def pair(p, carry):
            c0 = 2 * p
            for half, bset in enumerate(sets):
                c = c0 + half
                cpre = jnp.minimum(c + 2, NCH - 2 + half)
                gwait(bset)
                pk_issue(cpre, bset)
                compute(bset)
                wexp_issue(cpre, bset)
                scatter(bset)
                idx_wait(cpre, bset)
                vec_prep(bset)
                gstart(bset)
            return carry

        lax.fori_loop(0, NCH // 2, pair, 0)
        gwait(sets[0])
        gwait(sets[1])

        plsc.subcore_barrier()
        pltpu.sync_copy(
            acc.at[pl.ds(sbase, STRIPE)],
            out_h.at[cid, pl.ds(sbase, STRIPE)],
        )

    return body(x, rel_feat, packed, w2b)


def _tc_reduce(parts):
    BR = 1000

    def red(p_ref, o_ref):
        o_ref[...] = p_ref[0] + p_ref[1]

    return pl.pallas_call(
        red,
        grid=(NROW // BR,),
        in_specs=[pl.BlockSpec((NC, BR, D), lambda i: (0, i, 0))],
        out_specs=pl.BlockSpec((BR, D), lambda i: (i, 0)),
        out_shape=jax.ShapeDtypeStruct((NROW, D), jnp.float32),
    )(parts)


def kernel(x, rel_feat, batch_heads, batch_rels, batch_tails, batch_ids, weights):
    pad = NPAD - N_FACT
    zi = jnp.zeros((pad,), jnp.int32)

    def p(a):
        return jnp.concatenate([a, zi])

    zfull = jnp.zeros((NPAD,), jnp.int32)
    idx = jnp.stack([
        p(batch_heads), p(batch_rels), p(batch_ids), p(batch_tails),
        zfull, zfull, zfull, zfull,
    ])
    packed = idx.reshape(8, NPAD // CK, CK).transpose(1, 0, 2).reshape(8 * NPAD)
    wpad = jnp.concatenate([weights, jnp.zeros((pad,), jnp.float32)])
    w2b = jnp.broadcast_to(wpad[:, None], (NPAD, L)).reshape(NPAD * L)
    parts = _sc_gnn(x, rel_feat, packed, w2b)
    summed = _tc_reduce(parts)
    return summed[:N_ENT], summed[N_ENT:]


# speedup vs baseline: 1.0574x; 1.0574x over previous
import functools

import jax
import jax.numpy as jnp
from jax import lax
from jax.experimental import pallas as pl
from jax.experimental.pallas import tpu as pltpu
from jax.experimental.pallas import tpu_sc as plsc

N_ENT = 10000
NUM_REL = 200
BATCH = 10
N_FACT = 320000
D = 128

NC, NS, L = 2, 16, 16
NW = NC * NS
CK = 48
NCH = 210
NPAD = NW * NCH * CK
NROW = N_ENT + BATCH * NUM_REL
STRIPE = 752


def _sc_gnn(x, rel_feat, heads, rels, ids, tails, w2b):
    mesh = plsc.VectorSubcoreMesh(core_axis_name="c", subcore_axis_name="s")

    def buffer_set():
        return [
            pltpu.VMEM((CK, D), jnp.float32),
            pltpu.VMEM((CK, D), jnp.float32),
            pltpu.VMEM((CK,), jnp.int32),
            pltpu.VMEM((CK * L,), jnp.float32),
            pltpu.VMEM((CK,), jnp.int32),
            pltpu.VMEM((CK,), jnp.int32),
            pltpu.VMEM((CK,), jnp.int32),
            pltpu.VMEM((CK,), jnp.int32),
            pltpu.VMEM((CK,), jnp.int32),
            pltpu.VMEM((CK,), jnp.int32),
            pltpu.SemaphoreType.DMA,
            pltpu.SemaphoreType.DMA,
        ]

    @functools.partial(
        pl.kernel,
        out_type=jax.ShapeDtypeStruct((NC, NROW, D), jnp.float32),
        mesh=mesh,
        scratch_types=[pltpu.VMEM_SHARED((NROW, D), jnp.float32)]
        + buffer_set() + buffer_set(),
    )
    def body(x_h, rf_h, hd_h, rl_h, id_h, tl_h, w2_h, out_h, acc, *bufs):
        sets = (bufs[:12], bufs[12:])
        cid = lax.axis_index("c")
        sid = lax.axis_index("s")
        wid = cid * NS + sid

        def pk_issue(c, bset):
            hv, rv, iv, tl, sem_i = bset[2], bset[7], bset[8], bset[9], bset[10]
            base = (wid * NCH + c) * CK
            pltpu.async_copy(hd_h.at[pl.ds(base, CK)], hv, sem_i)
            pltpu.async_copy(rl_h.at[pl.ds(base, CK)], rv, sem_i)
            pltpu.async_copy(id_h.at[pl.ds(base, CK)], iv, sem_i)
            pltpu.async_copy(tl_h.at[pl.ds(base, CK)], tl, sem_i)

        def wexp_issue(c, bset):
            wexp, sem_i = bset[3], bset[10]
            ch = wid * NCH + c
            pltpu.async_copy(w2_h.at[pl.ds(ch * CK * L, CK * L)], wexp, sem_i)

        def idx_wait(c, bset):
            hv, rv, iv, tl, wexp, sem_i = (
                bset[2], bset[7], bset[8], bset[9], bset[3], bset[10])
            base = (wid * NCH + c) * CK
            pltpu.make_async_copy(hd_h.at[pl.ds(base, CK)], hv, sem_i).wait()
            pltpu.make_async_copy(rl_h.at[pl.ds(base, CK)], rv, sem_i).wait()
            pltpu.make_async_copy(id_h.at[pl.ds(base, CK)], iv, sem_i).wait()
            pltpu.make_async_copy(tl_h.at[pl.ds(base, CK)], tl, sem_i).wait()
            pltpu.make_async_copy(
                w2_h.at[pl.ds((wid * NCH + c) * CK * L, CK * L)], wexp, sem_i).wait()

        def vec_prep(bset):
            rv, iv, tl, riv, rsv, tv = (
                bset[7], bset[8], bset[9], bset[4], bset[5], bset[6])
            for j in range(CK // L):
                sl = pl.ds(L * j, L)
                r16 = rv[sl] + iv[sl] * NUM_REL
                riv[sl] = r16
                rsv[sl] = r16 + N_ENT
                tv[sl] = tl[sl]

        def gstart(bset):
            xrows, rrows, riv, hv, sem_g = bset[0], bset[1], bset[4], bset[2], bset[11]
            pltpu.async_copy(x_h.at[hv], xrows, sem_g)
            pltpu.async_copy(rf_h.at[riv], rrows, sem_g)

        def gwait(bset):
            xrows, rrows, riv, hv, sem_g = bset[0], bset[1], bset[4], bset[2], bset[11]
            pltpu.make_async_copy(x_h.at[hv], xrows, sem_g).wait()
            pltpu.make_async_copy(rf_h.at[riv], rrows, sem_g).wait()

        def compute(bset):
            xrows, rrows, wexp = bset[0], bset[1], bset[3]

            def fact(f, c2):
                s = wexp[pl.ds(f * L, L)]
                s2 = s * s
                for j in range(D // L):
                    sl = pl.ds(L * j, L)
                    xrows[f, sl] = (xrows[f, sl] + rrows[f, sl]) * s2
                return c2

            lax.fori_loop(0, CK, fact, 0, unroll=4)

        def scatter(bset):
            xrows, rsv, tv = bset[0], bset[5], bset[6]
            pltpu.sync_copy(xrows, acc.at[tv], add=True)
            pltpu.sync_copy(xrows, acc.at[rsv], add=True)

        xrows0 = sets[0][0]
        zvec = jnp.zeros((L,), jnp.float32)

        def zrow(r, carry):
            for j in range(D // L):
                xrows0[r, pl.ds(L * j, L)] = zvec
            return carry

        lax.fori_loop(0, CK, zrow, 0)
        sbase = jnp.minimum(sid * STRIPE, NROW - STRIPE)
        for kk in range(STRIPE // CK):
            pltpu.sync_copy(xrows0, acc.at[pl.ds(sbase + kk * CK, CK)])
        rem = STRIPE % CK
        if rem:
            pltpu.sync_copy(
                xrows0.at[pl.ds(0, rem)],
                acc.at[pl.ds(sbase + (STRIPE // CK) * CK, rem)],
            )
        plsc.subcore_barrier()

        for half, bset in enumerate(sets):
            pk_issue(half, bset)
            wexp_issue(half, bset)
            idx_wait(half, bset)
            vec_prep(bset)
            gstart(bset)

        def pair(p, carry):
            c0 = 2 * p
            for half, bset in enumerate(sets):
                c = c0 + half
                cpre = jnp.minimum(c + 2, NCH - 2 + half)
                gwait(bset)
                pk_issue(cpre, bset)
                compute(bset)
                wexp_issue(cpre, bset)
                scatter(bset)
                idx_wait(cpre, bset)
                vec_prep(bset)
                gstart(bset)
            return carry

        lax.fori_loop(0, NCH // 2, pair, 0)
        gwait(sets[0])
        gwait(sets[1])

        plsc.subcore_barrier()
        pltpu.sync_copy(
            acc.at[pl.ds(sbase, STRIPE)],
            out_h.at[cid, pl.ds(sbase, STRIPE)],
        )

    return body(x, rel_feat, heads, rels, ids, tails, w2b)


def _tc_reduce(parts):
    BR = 1000

    def red(p_ref, o_ref):
        o_ref[...] = p_ref[0] + p_ref[1]

    return pl.pallas_call(
        red,
        grid=(NROW // BR,),
        in_specs=[pl.BlockSpec((NC, BR, D), lambda i: (0, i, 0))],
        out_specs=pl.BlockSpec((BR, D), lambda i: (i, 0)),
        out_shape=jax.ShapeDtypeStruct((NROW, D), jnp.float32),
    )(parts)


def kernel(x, rel_feat, batch_heads, batch_rels, batch_tails, batch_ids, weights):
    pad = NPAD - N_FACT
    zi = jnp.zeros((pad,), jnp.int32)

    def p(a):
        return jnp.concatenate([a, zi])

    wpad = jnp.concatenate([weights, jnp.zeros((pad,), jnp.float32)])
    w2b = jnp.broadcast_to(wpad[:, None], (NPAD, L)).reshape(NPAD * L)
    parts = _sc_gnn(x, rel_feat, p(batch_heads), p(batch_rels), p(batch_ids),
                    p(batch_tails), w2b)
    summed = _tc_reduce(parts)
    return summed[:N_ENT], summed[N_ENT:]


# speedup vs baseline: 1.8062x; 1.7081x over previous
import functools

import jax
import jax.numpy as jnp
from jax import lax
from jax.experimental import pallas as pl
from jax.experimental.pallas import tpu as pltpu
from jax.experimental.pallas import tpu_sc as plsc

N_ENT = 10000
NUM_REL = 200
BATCH = 10
N_FACT = 320000
D = 128

NC, NS, L = 2, 16, 16
NW = NC * NS
CK = 48
NCH = 210
NPAD = NW * NCH * CK
NROW = N_ENT + BATCH * NUM_REL
STRIPE = 752


def _sc_gnn(x, rel_feat, heads, rels, ids, tails, w):
    mesh = plsc.VectorSubcoreMesh(core_axis_name="c", subcore_axis_name="s")

    def buffer_set():
        return [
            pltpu.VMEM((CK, D), jnp.float32),
            pltpu.VMEM((CK, D), jnp.float32),
            pltpu.VMEM((CK,), jnp.int32),
            pltpu.VMEM((CK,), jnp.int32),
            pltpu.VMEM((CK,), jnp.int32),
            pltpu.VMEM((CK,), jnp.int32),
            pltpu.VMEM((CK,), jnp.float32),
            pltpu.VMEM((CK,), jnp.int32),
            pltpu.VMEM((CK,), jnp.int32),
            pltpu.VMEM((CK,), jnp.int32),
            pltpu.VMEM((CK + L,), jnp.float32),
            pltpu.SemaphoreType.DMA,
            pltpu.SemaphoreType.DMA,
            pltpu.SemaphoreType.DMA,
        ]

    @functools.partial(
        pl.kernel,
        out_type=jax.ShapeDtypeStruct((NC, NROW, D), jnp.float32),
        mesh=mesh,
        scratch_types=[pltpu.VMEM_SHARED((NROW, D), jnp.float32)]
        + buffer_set() + buffer_set(),
    )
    def body(x_h, rf_h, hd_h, rl_h, id_h, tl_h, w_h, out_h, acc, *bufs):
        sets = (bufs[:14], bufs[14:])
        cid = lax.axis_index("c")
        sid = lax.axis_index("s")
        wid = cid * NS + sid

        def idx_copies(c, bset):
            hv, rv, iv, tl, wraw, sem_i = bset[2], bset[3], bset[4], bset[5], bset[6], bset[11]
            base = (wid * NCH + c) * CK
            return [
                pltpu.async_copy(hd_h.at[pl.ds(base, CK)], hv, sem_i),
                pltpu.async_copy(rl_h.at[pl.ds(base, CK)], rv, sem_i),
                pltpu.async_copy(id_h.at[pl.ds(base, CK)], iv, sem_i),
                pltpu.async_copy(tl_h.at[pl.ds(base, CK)], tl, sem_i),
                pltpu.async_copy(w_h.at[pl.ds(base, CK)], wraw, sem_i),
            ]

        def idx_issue(c, bset):
            idx_copies(c, bset)

        def idx_wait(c, bset):
            hv, rv, iv, tl, wraw, sem_i = bset[2], bset[3], bset[4], bset[5], bset[6], bset[11]
            base = (wid * NCH + c) * CK
            pltpu.make_async_copy(hd_h.at[pl.ds(base, CK)], hv, sem_i).wait()
            pltpu.make_async_copy(rl_h.at[pl.ds(base, CK)], rv, sem_i).wait()
            pltpu.make_async_copy(id_h.at[pl.ds(base, CK)], iv, sem_i).wait()
            pltpu.make_async_copy(tl_h.at[pl.ds(base, CK)], tl, sem_i).wait()
            pltpu.make_async_copy(w_h.at[pl.ds(base, CK)], wraw, sem_i).wait()

        def vec_prep(bset):
            rv, iv, tl, wraw, riv, rsv, tv, w2v = (
                bset[3], bset[4], bset[5], bset[6], bset[7], bset[8], bset[9], bset[10])
            for j in range(CK // L):
                sl = pl.ds(L * j, L)
                r16 = rv[sl] + iv[sl] * NUM_REL
                riv[sl] = r16
                rsv[sl] = r16 + N_ENT
                tv[sl] = tl[sl]
                w16 = wraw[sl]
                w2v[sl] = w16 * w16

        def gstart(bset):
            xrows, rrows, hv, riv, sem_g = bset[0], bset[1], bset[2], bset[7], bset[12]
            pltpu.async_copy(x_h.at[hv], xrows, sem_g)
            pltpu.async_copy(rf_h.at[riv], rrows, sem_g)

        def gwait(bset):
            xrows, rrows, hv, riv, sem_g = bset[0], bset[1], bset[2], bset[7], bset[12]
            pltpu.make_async_copy(x_h.at[hv], xrows, sem_g).wait()
            pltpu.make_async_copy(rf_h.at[riv], rrows, sem_g).wait()

        def compute(bset):
            xrows, rrows, w2v = bset[0], bset[1], bset[10]

            def group(g, c2):
                w16 = w2v[pl.ds(L * g, L)]
                for t in range(L):
                    s = w16[t]
                    f = L * g + t
                    for j in range(D // L):
                        sl = pl.ds(L * j, L)
                        xrows[f, sl] = (xrows[f, sl] + rrows[f, sl]) * s
                return c2

            lax.fori_loop(0, CK // L, group, 0)

        def scatter(bset):
            xrows, rsv, tv = bset[0], bset[8], bset[9]
            pltpu.sync_copy(xrows, acc.at[tv], add=True)
            pltpu.sync_copy(xrows, acc.at[rsv], add=True)

        xrows0 = sets[0][0]
        zvec = jnp.zeros((L,), jnp.float32)

        def zrow(r, carry):
            for j in range(D // L):
                xrows0[r, pl.ds(L * j, L)] = zvec
            return carry

        lax.fori_loop(0, CK, zrow, 0)
        sbase = jnp.minimum(sid * STRIPE, NROW - STRIPE)
        for kk in range(STRIPE // CK):
            pltpu.sync_copy(xrows0, acc.at[pl.ds(sbase + kk * CK, CK)])
        rem = STRIPE % CK
        if rem:
            pltpu.sync_copy(
                xrows0.at[pl.ds(0, rem)],
                acc.at[pl.ds(sbase + (STRIPE // CK) * CK, rem)],
            )
        plsc.subcore_barrier()

        for half, bset in enumerate(sets):
            idx_issue(half, bset)
            idx_wait(half, bset)
            vec_prep(bset)
            gstart(bset)

        def pair(p, carry):
            c0 = 2 * p
            for half, bset in enumerate(sets):
                c = c0 + half
                cpre = jnp.minimum(c + 2, NCH - 2 + half)
                gwait(bset)
                idx_issue(cpre, bset)
                compute(bset)
                scatter(bset)
                idx_wait(cpre, bset)
                vec_prep(bset)
                gstart(bset)
            return carry

        lax.fori_loop(0, NCH // 2, pair, 0)
        gwait(sets[0])
        gwait(sets[1])

        plsc.subcore_barrier()
        pltpu.sync_copy(
            acc.at[pl.ds(sbase, STRIPE)],
            out_h.at[cid, pl.ds(sbase, STRIPE)],
        )

    return body(x, rel_feat, heads, rels, ids, tails, w)


def _tc_reduce(parts):
    BR = 1000

    def red(p_ref, o_ref):
        o_ref[...] = p_ref[0] + p_ref[1]

    return pl.pallas_call(
        red,
        grid=(NROW // BR,),
        in_specs=[pl.BlockSpec((NC, BR, D), lambda i: (0, i, 0))],
        out_specs=pl.BlockSpec((BR, D), lambda i: (i, 0)),
        out_shape=jax.ShapeDtypeStruct((NROW, D), jnp.float32),
    )(parts)


def kernel(x, rel_feat, batch_heads, batch_rels, batch_tails, batch_ids, weights):
    pad = NPAD - N_FACT
    zi = jnp.zeros((pad,), jnp.int32)
    heads = jnp.concatenate([batch_heads, zi])
    rels = jnp.concatenate([batch_rels, zi])
    ids = jnp.concatenate([batch_ids, zi])
    tails = jnp.concatenate([batch_tails, zi])
    w = jnp.concatenate([weights, jnp.zeros((pad,), jnp.float32)])
    parts = _sc_gnn(x, rel_feat, heads, rels, ids, tails, w)
    summed = _tc_reduce(parts)
    return summed[:N_ENT], summed[N_ENT:]


# speedup vs baseline: 1.8386x; 1.0179x over previous
import functools

import jax
import jax.numpy as jnp
from jax import lax
from jax.experimental import pallas as pl
from jax.experimental.pallas import tpu as pltpu
from jax.experimental.pallas import tpu_sc as plsc

N_ENT = 10000
NUM_REL = 200
BATCH = 10
N_FACT = 320000
D = 128

NC, NS, L = 2, 16, 16
NW = NC * NS
CK = 48
NCH = 210
NPAD = NW * NCH * CK
NROW = N_ENT + BATCH * NUM_REL
STRIPE = 752


def _sc_gnn(x, rel_feat, heads, rels, ids, tails, w):
    mesh = plsc.VectorSubcoreMesh(core_axis_name="c", subcore_axis_name="s")

    def buffer_set():
        return [
            pltpu.VMEM((CK, D), jnp.float32),
            pltpu.VMEM((CK, D), jnp.float32),
            pltpu.VMEM((CK,), jnp.int32),
            pltpu.VMEM((CK,), jnp.int32),
            pltpu.VMEM((CK,), jnp.int32),
            pltpu.VMEM((CK,), jnp.int32),
            pltpu.VMEM((CK,), jnp.float32),
            pltpu.VMEM((CK,), jnp.int32),
            pltpu.VMEM((CK,), jnp.int32),
            pltpu.VMEM((CK,), jnp.int32),
            pltpu.VMEM((CK + L,), jnp.float32),
            pltpu.SemaphoreType.DMA,
            pltpu.SemaphoreType.DMA,
            pltpu.SemaphoreType.DMA,
        ]

    @functools.partial(
        pl.kernel,
        out_type=jax.ShapeDtypeStruct((NC, NROW, D), jnp.float32),
        mesh=mesh,
        scratch_types=[pltpu.VMEM_SHARED((NROW, D), jnp.float32)]
        + buffer_set() + buffer_set(),
    )
    def body(x_h, rf_h, hd_h, rl_h, id_h, tl_h, w_h, out_h, acc, *bufs):
        sets = (bufs[:14], bufs[14:])
        cid = lax.axis_index("c")
        sid = lax.axis_index("s")
        wid = cid * NS + sid

        def idx_copies(c, bset):
            hv, rv, iv, tl, wraw, sem_i = bset[2], bset[3], bset[4], bset[5], bset[6], bset[11]
            base = (wid * NCH + c) * CK
            return [
                pltpu.async_copy(hd_h.at[pl.ds(base, CK)], hv, sem_i),
                pltpu.async_copy(rl_h.at[pl.ds(base, CK)], rv, sem_i),
                pltpu.async_copy(id_h.at[pl.ds(base, CK)], iv, sem_i),
                pltpu.async_copy(tl_h.at[pl.ds(base, CK)], tl, sem_i),
                pltpu.async_copy(w_h.at[pl.ds(base, CK)], wraw, sem_i),
            ]

        def idx_issue(c, bset):
            idx_copies(c, bset)

        def idx_wait(c, bset):
            hv, rv, iv, tl, wraw, sem_i = bset[2], bset[3], bset[4], bset[5], bset[6], bset[11]
            base = (wid * NCH + c) * CK
            pltpu.make_async_copy(hd_h.at[pl.ds(base, CK)], hv, sem_i).wait()
            pltpu.make_async_copy(rl_h.at[pl.ds(base, CK)], rv, sem_i).wait()
            pltpu.make_async_copy(id_h.at[pl.ds(base, CK)], iv, sem_i).wait()
            pltpu.make_async_copy(tl_h.at[pl.ds(base, CK)], tl, sem_i).wait()
            pltpu.make_async_copy(w_h.at[pl.ds(base, CK)], wraw, sem_i).wait()

        def vec_prep(bset):
            rv, iv, tl, wraw, riv, rsv, tv, w2v = (
                bset[3], bset[4], bset[5], bset[6], bset[7], bset[8], bset[9], bset[10])
            for j in range(CK // L):
                sl = pl.ds(L * j, L)
                r16 = rv[sl] + iv[sl] * NUM_REL
                riv[sl] = r16
                rsv[sl] = r16 + N_ENT
                tv[sl] = tl[sl]
                w16 = wraw[sl]
                w2v[sl] = w16 * w16

        def gstart(bset):
            xrows, rrows, hv, riv, sem_g = bset[0], bset[1], bset[2], bset[7], bset[12]
            pltpu.async_copy(x_h.at[hv], xrows, sem_g)
            pltpu.async_copy(rf_h.at[riv], rrows, sem_g)

        def gwait(bset):
            xrows, rrows, hv, riv, sem_g = bset[0], bset[1], bset[2], bset[7], bset[12]
            pltpu.make_async_copy(x_h.at[hv], xrows, sem_g).wait()
            pltpu.make_async_copy(rf_h.at[riv], rrows, sem_g).wait()

        def compute(bset):
            xrows, rrows, w2v = bset[0], bset[1], bset[10]

            def group(g, c2):
                w16 = w2v[pl.ds(L * g, L)]
                for t in range(L):
                    s = w16[t]
                    f = L * g + t
                    for j in range(D // L):
                        sl = pl.ds(L * j, L)
                        xrows[f, sl] = (xrows[f, sl] + rrows[f, sl]) * s
                return c2

            lax.fori_loop(0, CK // L, group, 0)

        def scatter(bset):
            xrows, rsv, tv, sem_s = bset[0], bset[8], bset[9], bset[13]
            cp = pltpu.async_copy(xrows, acc.at[tv], sem_s, add=True)
            pltpu.sync_copy(xrows, acc.at[rsv], add=True)
            cp.wait()

        xrows0 = sets[0][0]
        zvec = jnp.zeros((L,), jnp.float32)

        def zrow(r, carry):
            for j in range(D // L):
                xrows0[r, pl.ds(L * j, L)] = zvec
            return carry

        lax.fori_loop(0, CK, zrow, 0)
        sbase = jnp.minimum(sid * STRIPE, NROW - STRIPE)
        for kk in range(STRIPE // CK):
            pltpu.sync_copy(xrows0, acc.at[pl.ds(sbase + kk * CK, CK)])
        rem = STRIPE % CK
        if rem:
            pltpu.sync_copy(
                xrows0.at[pl.ds(0, rem)],
                acc.at[pl.ds(sbase + (STRIPE // CK) * CK, rem)],
            )
        plsc.subcore_barrier()

        for half, bset in enumerate(sets):
            idx_issue(half, bset)
            idx_wait(half, bset)
            vec_prep(bset)
            gstart(bset)

        def pair(p, carry):
            c0 = 2 * p
            for half, bset in enumerate(sets):
                c = c0 + half
                cpre = jnp.minimum(c + 2, NCH - 2 + half)
                gwait(bset)
                idx_issue(cpre, bset)
                compute(bset)
                scatter(bset)
                idx_wait(cpre, bset)
                vec_prep(bset)
                gstart(bset)
            return carry

        lax.fori_loop(0, NCH // 2, pair, 0)
        gwait(sets[0])
        gwait(sets[1])

        plsc.subcore_barrier()
        pltpu.sync_copy(
            acc.at[pl.ds(sbase, STRIPE)],
            out_h.at[cid, pl.ds(sbase, STRIPE)],
        )

    return body(x, rel_feat, heads, rels, ids, tails, w)


def _tc_reduce(parts):
    BR = 1000

    def red(p_ref, o_ref):
        o_ref[...] = p_ref[0] + p_ref[1]

    return pl.pallas_call(
        red,
        grid=(NROW // BR,),
        in_specs=[pl.BlockSpec((NC, BR, D), lambda i: (0, i, 0))],
        out_specs=pl.BlockSpec((BR, D), lambda i: (i, 0)),
        out_shape=jax.ShapeDtypeStruct((NROW, D), jnp.float32),
    )(parts)


def kernel(x, rel_feat, batch_heads, batch_rels, batch_tails, batch_ids, weights):
    pad = NPAD - N_FACT
    zi = jnp.zeros((pad,), jnp.int32)
    heads = jnp.concatenate([batch_heads, zi])
    rels = jnp.concatenate([batch_rels, zi])
    ids = jnp.concatenate([batch_ids, zi])
    tails = jnp.concatenate([batch_tails, zi])
    w = jnp.concatenate([weights, jnp.zeros((pad,), jnp.float32)])
    parts = _sc_gnn(x, rel_feat, heads, rels, ids, tails, w)
    summed = _tc_reduce(parts)
    return summed[:N_ENT], summed[N_ENT:]
